# baseline (speedup 1.0000x reference)
import jax
import jax.numpy as jnp
from jax.experimental import pallas as pl
from jax.experimental.pallas import tpu as pltpu


def _copy_body(pos_ref, out_ref):
    out_ref[...] = pos_ref[...]


def kernel(x, pos_table):
    del x
    t = pos_table.T
    out_t = pl.pallas_call(
        _copy_body,
        grid=(2,),
        in_specs=[pl.BlockSpec((32, 200), lambda i: (i, 0))],
        out_specs=pl.BlockSpec((32, 200), lambda i: (i, 0)),
        out_shape=jax.ShapeDtypeStruct(t.shape, t.dtype),
        compiler_params=pltpu.CompilerParams(
            dimension_semantics=("parallel",),
            disable_bounds_checks=True,
            disable_semaphore_checks=True,
            skip_device_barrier=True,
        ),
    )(t)
    return out_t.T


# speedup vs baseline: 1.0223x; 1.0223x over previous
import jax
import jax.numpy as jnp
from jax.experimental import pallas as pl


def _copy_body(pos_ref, out_ref):
    out_ref[...] = pos_ref[...]


def kernel(x, pos_table):
    del x
    t = pos_table.T
    out_t = pl.pallas_call(
        _copy_body,
        out_shape=jax.ShapeDtypeStruct(t.shape, t.dtype),
    )(t)
    return out_t.T
